# Initial kernel scaffold; baseline (speedup 1.0000x reference)
import jax
import jax.numpy as jnp
from jax.experimental import pallas as pl


def kernel(x):
    raise NotImplementedError("write your pallas kernel here")


# speedup vs baseline: 3.7474x; 3.7474x over previous
import functools
import math

import jax
import jax.numpy as jnp
from jax import lax
from jax.experimental import pallas as pl
from jax.experimental.pallas import tpu as pltpu
from jax.experimental.pallas import tpu_sc as plsc

B = 128
E = 8192
K = math.ceil(0.05 * E)

NC = 2
NS = 16
L = 16
NW = NC * NS
RPW = B // NW
NCHUNK = E // L

INT_MIN = -(2 ** 31)
ONE_BITS = 0x3F800000

_mesh = plsc.VectorSubcoreMesh(
    core_axis_name="c", subcore_axis_name="s", num_cores=NC, num_subcores=NS
)


def _build(interpret=False):
    return functools.partial(
        pl.kernel,
        out_type=jax.ShapeDtypeStruct((B * E,), jnp.int32),
        mesh=_mesh,
        scratch_types=[
            pltpu.VMEM((E,), jnp.int32),
            pltpu.VMEM((256 * L,), jnp.int32),
            pltpu.VMEM((256,), jnp.int32),
        ],
        compiler_params=pltpu.CompilerParams(needs_layout_passes=False),
        interpret=interpret,
    )(_kwta_body)


def _kwta_body(x_hbm, out_hbm, d_ref, hist_ref, tots_ref):
    wid = lax.axis_index("s") * NC + lax.axis_index("c")
    lanes = lax.iota(jnp.int32, L)
    zeros16 = lax.full((L,), 0, jnp.int32)
    ones16 = lax.full((L,), 1, jnp.int32)

    def _clear(i, c):
        hist_ref[pl.ds(i * L, L)] = zeros16
        return c

    lax.fori_loop(0, 256, _clear, 0)

    def _shrl(v, amount):
        return lax.shift_right_logical(v, lax.full_like(v, amount))

    def _row_body(ri, carry):
        row = wid * RPW + ri
        pltpu.sync_copy(x_hbm.at[pl.ds(row * E, E)], d_ref)

        def _fill0(i, c):
            v = d_ref[pl.ds(i * L, L)]
            neg = lax.shift_right_arithmetic(v, lax.full_like(v, 31))
            u = v ^ (neg | INT_MIN)
            d_ref[pl.ds(i * L, L)] = u
            byte = _shrl(u, 24)
            plsc.addupdate_scatter(hist_ref, [byte * L + lanes], ones16)
            return c

        lax.fori_loop(0, NCHUNK, _fill0, 0)

        prefix = jnp.int32(0)
        r = jnp.int32(K)

        for p in range(4):
            if p > 0:
                pshift = 32 - 8 * p
                bshift = 24 - 8 * p

                def _fillp(i, c, pshift=pshift, bshift=bshift, prefix=prefix):
                    u = d_ref[pl.ds(i * L, L)]
                    act = _shrl(u, pshift) == prefix
                    byte = _shrl(u, bshift) & 0xFF
                    plsc.addupdate_scatter(
                        hist_ref, [byte * L + lanes], ones16, mask=act
                    )
                    return c

                lax.fori_loop(0, NCHUNK, _fillp, 0)

            def _merge(j, c):
                acc = zeros16
                bidx = (j * 16 + lanes) * L
                for l in range(L):
                    acc = acc + plsc.load_gather(hist_ref, [bidx + l])
                for l in range(L):
                    hist_ref[pl.ds(j * 256 + l * L, L)] = zeros16
                tots_ref[pl.ds(j * 16, 16)] = acc
                return c

            lax.fori_loop(0, 16, _merge, 0)

            def _scan(jj, c):
                nb, above, base = c
                j = 15 - jj
                v = tots_ref[pl.ds(j * 16, 16)]
                rev = lax.rev(v, (0,))
                csum = plsc.cumsum(rev) + base
                ge = csum >= r
                nb = nb + jnp.sum(ge.astype(jnp.int32))
                above = above + jnp.sum(jnp.where(ge, 0, rev))
                base = base + jnp.sum(v)
                return nb, above, base

            nb, above, _ = lax.fori_loop(
                0, 16, _scan, (jnp.int32(0), jnp.int32(0), jnp.int32(0))
            )
            b_star = nb - 1
            prefix = (prefix << 8) | b_star if p > 0 else b_star
            r = r - above

        t_signed = prefix ^ INT_MIN

        def _mask(i, c):
            u = d_ref[pl.ds(i * L, L)]
            keep = (u ^ INT_MIN) >= t_signed
            d_ref[pl.ds(i * L, L)] = jnp.where(
                keep, jnp.int32(ONE_BITS), jnp.int32(0)
            )
            return c

        lax.fori_loop(0, NCHUNK, _mask, 0)
        pltpu.sync_copy(d_ref, out_hbm.at[pl.ds(row * E, E)])
        return carry

    lax.fori_loop(0, RPW, _row_body, 0)


_kwta_sc = _build()


def kernel(x):
    xbits = lax.bitcast_convert_type(x, jnp.int32).reshape(-1)
    out = _kwta_sc(xbits)
    return lax.bitcast_convert_type(out.reshape(x.shape), jnp.float32)


# speedup vs baseline: 4.9300x; 1.3156x over previous
import functools
import math

import jax
import jax.numpy as jnp
from jax import lax
from jax.experimental import pallas as pl
from jax.experimental.pallas import tpu as pltpu
from jax.experimental.pallas import tpu_sc as plsc

B = 128
E = 8192
K = math.ceil(0.05 * E)

NC = 2
NS = 16
L = 16
NW = NC * NS
RPW = B // NW
NCHUNK = E // L
UNROLL = 4

INT_MIN = -(2 ** 31)
ONE_BITS = 0x3F800000

_mesh = plsc.VectorSubcoreMesh(
    core_axis_name="c", subcore_axis_name="s", num_cores=NC, num_subcores=NS
)


def _build(interpret=False):
    return functools.partial(
        pl.kernel,
        out_type=jax.ShapeDtypeStruct((B * E,), jnp.int32),
        mesh=_mesh,
        scratch_types=[
            pltpu.VMEM((E,), jnp.int32),
            pltpu.VMEM((E,), jnp.int32),
            pltpu.VMEM((256 * L,), jnp.int32),
            pltpu.VMEM((256,), jnp.int32),
        ],
        compiler_params=pltpu.CompilerParams(needs_layout_passes=False),
        interpret=interpret,
    )(_kwta_body)


def _kwta_body(x_hbm, out_hbm, d_ref, cand_ref, hist_ref, tots_ref):
    wid = lax.axis_index("s") * NC + lax.axis_index("c")
    lanes = lax.iota(jnp.int32, L)
    zeros16 = lax.full((L,), 0, jnp.int32)
    ones16 = lax.full((L,), 1, jnp.int32)

    def _clear(i, c):
        hist_ref[pl.ds(i * L, L)] = zeros16
        return c

    lax.fori_loop(0, 256, _clear, 0)

    def _shrl(v, amount):
        return lax.shift_right_logical(v, lax.full_like(v, amount))

    def _merge256_and_scan(r):
        def _merge(j, c):
            acc = zeros16
            bidx = (j * 16 + lanes) * L
            for l in range(L):
                acc = acc + plsc.load_gather(hist_ref, [bidx + l])
            for l in range(L):
                hist_ref[pl.ds(j * 256 + l * L, L)] = zeros16
            tots_ref[pl.ds(j * 16, 16)] = acc
            return c

        lax.fori_loop(0, 16, _merge, 0)

        def _scan(jj, c):
            nb, above, base = c
            j = 15 - jj
            v = tots_ref[pl.ds(j * 16, 16)]
            rev = lax.rev(v, (0,))
            csum = plsc.cumsum(rev) + base
            ge = csum >= r
            nb = nb + jnp.sum(ge.astype(jnp.int32))
            above = above + jnp.sum(jnp.where(ge, 0, rev))
            base = base + jnp.sum(v)
            return nb, above, base

        nb, above, _ = lax.fori_loop(
            0, 16, _scan, (jnp.int32(0), jnp.int32(0), jnp.int32(0))
        )
        return nb - 1, above

    def _row_body(ri, carry):
        row = wid * RPW + ri
        pltpu.sync_copy(x_hbm.at[pl.ds(row * E, E)], d_ref)

        def _fill0(i, c):
            for s in range(UNROLL):
                off = (i * UNROLL + s) * L
                v = d_ref[pl.ds(off, L)]
                neg = lax.shift_right_arithmetic(v, lax.full_like(v, 31))
                u = v ^ (neg | INT_MIN)
                d_ref[pl.ds(off, L)] = u
                byte = _shrl(u, 24)
                plsc.addupdate_scatter(hist_ref, [byte * L + lanes], ones16)
            return c

        lax.fori_loop(0, NCHUNK // UNROLL, _fill0, 0)

        b1, above = _merge256_and_scan(jnp.int32(K))
        r = jnp.int32(K) - above

        def _fill1(i, cnt):
            for s in range(UNROLL):
                off = (i * UNROLL + s) * L
                u = d_ref[pl.ds(off, L)]
                act = _shrl(u, 24) == b1
                byte = _shrl(u, 16) & 0xFF
                plsc.addupdate_scatter(
                    hist_ref, [byte * L + lanes], ones16, mask=act
                )
                plsc.store_scatter(
                    cand_ref, [cnt * L + lanes], u, mask=act
                )
                cnt = cnt + act.astype(jnp.int32)
            return cnt

        cnt = lax.fori_loop(0, NCHUNK // UNROLL, _fill1, zeros16)

        b2, above = _merge256_and_scan(r)
        prefix = (b1 << 8) | b2
        r = r - above
        m = lax.reduce_max(cnt, axes=(0,))

        for q in range(4):
            pshift = 16 - 4 * q
            bshift = pshift - 4

            def _fillq(j, c, pshift=pshift, bshift=bshift, prefix=prefix):
                u = cand_ref[pl.ds(j * L, L)]
                act = (_shrl(u, pshift) == prefix) & (j < cnt)
                nib = _shrl(u, bshift) & 0xF
                plsc.addupdate_scatter(
                    hist_ref, [nib * L + lanes], ones16, mask=act
                )
                return c

            lax.fori_loop(0, m, _fillq, 0)

            acc = zeros16
            for l in range(L):
                acc = acc + plsc.load_gather(hist_ref, [lanes * L + l])
            for l in range(L):
                hist_ref[pl.ds(l * L, L)] = zeros16
            rev = lax.rev(acc, (0,))
            csum = plsc.cumsum(rev)
            ge = csum >= r
            nb = jnp.sum(ge.astype(jnp.int32))
            above = jnp.sum(jnp.where(ge, 0, rev))
            prefix = (prefix << 4) | (nb - 1)
            r = r - above

        t_signed = prefix ^ INT_MIN

        def _mask(i, c):
            for s in range(UNROLL):
                off = (i * UNROLL + s) * L
                u = d_ref[pl.ds(off, L)]
                keep = (u ^ INT_MIN) >= t_signed
                d_ref[pl.ds(off, L)] = jnp.where(
                    keep, jnp.int32(ONE_BITS), jnp.int32(0)
                )
            return c

        lax.fori_loop(0, NCHUNK // UNROLL, _mask, 0)
        pltpu.sync_copy(d_ref, out_hbm.at[pl.ds(row * E, E)])
        return carry

    lax.fori_loop(0, RPW, _row_body, 0)


_kwta_sc = _build()


def kernel(x):
    xbits = lax.bitcast_convert_type(x, jnp.int32).reshape(-1)
    out = _kwta_sc(xbits)
    return lax.bitcast_convert_type(out.reshape(x.shape), jnp.float32)
